# baseline (speedup 1.0000x reference)
import functools

import jax
import jax.numpy as jnp
from jax import lax
from jax.experimental import pallas as pl
from jax.experimental.pallas import tpu as pltpu
from jax.experimental.pallas import tpu_sc as plsc

NUM_CORES = 2
NUM_SUBCORES = 16
NW = NUM_CORES * NUM_SUBCORES
RING = 16


def _sc_gather_pair(user_id, item_id, user_table, item_table):
    B = user_id.shape[0]
    D = user_table.shape[1]
    bpw = B // NW
    mesh = plsc.VectorSubcoreMesh(core_axis_name="c", subcore_axis_name="s")

    @functools.partial(
        pl.kernel,
        mesh=mesh,
        out_type=[
            jax.ShapeDtypeStruct((B, D), jnp.float32),
            jax.ShapeDtypeStruct((B, D), jnp.float32),
        ],
        scratch_types=[
            pltpu.VMEM((bpw,), jnp.int32),
            pltpu.VMEM((bpw,), jnp.int32),
            pltpu.SemaphoreType.DMA,
            pltpu.SemaphoreType.DMA,
        ],
    )
    def gather_kernel(uid_hbm, iid_hbm, ut_hbm, it_hbm, uout_hbm, iout_hbm,
                      uidx_v, iidx_v, sem_u, sem_i):
        wid = lax.axis_index("s") * NUM_CORES + lax.axis_index("c")
        base = wid * bpw
        pltpu.sync_copy(uid_hbm.at[pl.ds(base, bpw)], uidx_v)
        pltpu.sync_copy(iid_hbm.at[pl.ds(base, bpw)], iidx_v)

        def group(g, carry):
            uvec = uidx_v[pl.ds(g * 16, 16)]
            ivec = iidx_v[pl.ds(g * 16, 16)]
            for j in range(16):
                pltpu.async_copy(ut_hbm.at[pl.ds(uvec[j], 1)],
                                 uout_hbm.at[pl.ds(base + g * 16 + j, 1)],
                                 sem_u)
                pltpu.async_copy(it_hbm.at[pl.ds(ivec[j], 1)],
                                 iout_hbm.at[pl.ds(base + g * 16 + j, 1)],
                                 sem_i)

            @pl.when(g >= 1)
            def _throttle():
                for _ in range(16):
                    pltpu.make_async_copy(
                        ut_hbm.at[pl.ds(0, 1)],
                        uout_hbm.at[pl.ds(base, 1)], sem_u).wait()
                    pltpu.make_async_copy(
                        it_hbm.at[pl.ds(0, 1)],
                        iout_hbm.at[pl.ds(base, 1)], sem_i).wait()

            return carry

        lax.fori_loop(0, bpw // 16, group, 0)

        for _ in range(16):
            pltpu.make_async_copy(ut_hbm.at[pl.ds(0, 1)],
                                  uout_hbm.at[pl.ds(base, 1)], sem_u).wait()
            pltpu.make_async_copy(it_hbm.at[pl.ds(0, 1)],
                                  iout_hbm.at[pl.ds(base, 1)], sem_i).wait()

    return gather_kernel(user_id, item_id, user_table, item_table)


def _mlp_body(u_ref, i_ref, p_ref, w1_ref, b1_ref, w2_ref, b2_ref, o_ref):
    w1 = w1_ref[...]
    D = u_ref.shape[1]
    h = (jnp.dot(u_ref[...], w1[0:D, :], precision=lax.Precision.HIGHEST,
                 preferred_element_type=jnp.float32)
         + jnp.dot(i_ref[...], w1[D:2 * D, :], precision=lax.Precision.HIGHEST,
                   preferred_element_type=jnp.float32)
         + p_ref[...] * w1[2 * D:2 * D + 1, :]
         + b1_ref[...])
    h = jnp.maximum(h, 0.0)
    z = jnp.dot(h, w2_ref[...], precision=lax.Precision.HIGHEST,
                preferred_element_type=jnp.float32) + b2_ref[...]
    o_ref[...] = jax.nn.sigmoid(z)


def kernel(user_id, item_id, price, user_table, item_table, W1, b1, W2, b2):
    B = user_id.shape[0]
    D = user_table.shape[1]
    H = W1.shape[1]
    u_emb, i_emb = _sc_gather_pair(user_id, item_id, user_table, item_table)

    blk = 2048
    grid = (B // blk,)
    out = pl.pallas_call(
        _mlp_body,
        grid=grid,
        in_specs=[
            pl.BlockSpec((blk, D), lambda i: (i, 0)),
            pl.BlockSpec((blk, D), lambda i: (i, 0)),
            pl.BlockSpec((blk, 1), lambda i: (i, 0)),
            pl.BlockSpec((2 * D + 1, H), lambda i: (0, 0)),
            pl.BlockSpec((1, H), lambda i: (0, 0)),
            pl.BlockSpec((H, 1), lambda i: (0, 0)),
            pl.BlockSpec((1, 1), lambda i: (0, 0)),
        ],
        out_specs=pl.BlockSpec((blk, 1), lambda i: (i, 0)),
        out_shape=jax.ShapeDtypeStruct((B, 1), jnp.float32),
    )(u_emb, i_emb, price.reshape(B, 1), W1, b1.reshape(1, H),
      W2, b2.reshape(1, 1))
    return out.reshape(B)


# speedup vs baseline: 7.4249x; 7.4249x over previous
import functools

import jax
import jax.numpy as jnp
from jax import lax
from jax.experimental import pallas as pl
from jax.experimental.pallas import tpu as pltpu
from jax.experimental.pallas import tpu_sc as plsc

NUM_CORES = 2
NUM_SUBCORES = 16
NW = NUM_CORES * NUM_SUBCORES
CBW = 65536
STRIPE = CBW // 8
CHUNK = 128


def _precompute_body(*refs):
    xs, wb_ref, o_ref = refs[:8], refs[8], refs[9]
    x = jnp.concatenate([r[...] for r in xs], axis=0)
    o_ref[...] = jax.lax.dot_general(
        x.astype(jnp.bfloat16), wb_ref[...].astype(jnp.bfloat16),
        dimension_numbers=(((0,), (0,)), ((), ())),
        preferred_element_type=jnp.float32)


def _precompute_packed(table_t, w):
    n = table_t.shape[1]
    nblk = pl.cdiv(n, CBW)
    wb = jnp.kron(jnp.eye(8, dtype=w.dtype), w)
    last = pl.cdiv(n, STRIPE) - 1
    stripe_specs = [
        pl.BlockSpec((32, STRIPE),
                     lambda i, j=j: (0, jnp.minimum(i * 8 + j, last)))
        for j in range(8)
    ]
    return pl.pallas_call(
        _precompute_body,
        grid=(nblk,),
        in_specs=stripe_specs + [pl.BlockSpec((256, 128), lambda i: (0, 0))],
        out_specs=pl.BlockSpec((STRIPE, 128), lambda i: (i, 0)),
        out_shape=jax.ShapeDtypeStruct((nblk * STRIPE, 128), jnp.float32),
    )(*([table_t] * 8), wb)


def _sc_gather_mlp(user_id, item_id, price, t_u, t_i, params):
    B = user_id.shape[0]
    bpw = B // NW
    nchunk = bpw // CHUNK
    mesh = plsc.VectorSubcoreMesh(core_axis_name="c", subcore_axis_name="s")

    @functools.partial(
        pl.kernel,
        mesh=mesh,
        compiler_params=pltpu.CompilerParams(needs_layout_passes=False),
        out_type=jax.ShapeDtypeStruct((B,), jnp.float32),
        scratch_types=[
            pltpu.VMEM((bpw,), jnp.int32),
            pltpu.VMEM((bpw,), jnp.int32),
            pltpu.VMEM((bpw,), jnp.float32),
            pltpu.VMEM((nchunk, CHUNK), jnp.int32),
            pltpu.VMEM((nchunk, CHUNK), jnp.int32),
            pltpu.VMEM((bpw,), jnp.int32),
            pltpu.VMEM((bpw,), jnp.int32),
            pltpu.VMEM((CHUNK, 128), jnp.float32),
            pltpu.VMEM((CHUNK, 128), jnp.float32),
            pltpu.VMEM((bpw,), jnp.float32),
            pltpu.VMEM((64,), jnp.float32),
            pltpu.SemaphoreType.DMA,
            pltpu.SemaphoreType.DMA,
        ],
    )
    def body(uid_hbm, iid_hbm, price_hbm, tu_hbm, ti_hbm, par_hbm, out_hbm,
             uid_v, iid_v, pr_v, gu_v, gi_v, cu_v, ci_v,
             su_v, si_v, z_v, par_v, sem_u, sem_i):
        wid = lax.axis_index("s") * NUM_CORES + lax.axis_index("c")
        base = wid * bpw
        pltpu.sync_copy(uid_hbm.at[pl.ds(base, bpw)], uid_v)
        pltpu.sync_copy(iid_hbm.at[pl.ds(base, bpw)], iid_v)
        pltpu.sync_copy(price_hbm.at[pl.ds(base, bpw)], pr_v)
        pltpu.sync_copy(par_hbm, par_v)
        w1p = par_v[pl.ds(0, 16)]
        b1 = par_v[pl.ds(16, 16)]
        w2 = par_v[pl.ds(32, 16)]
        b2 = par_v[pl.ds(48, 16)]
        iota = lax.iota(jnp.int32, 16)

        for c in range(nchunk):
            for s in range(CHUNK // 16):
                u = uid_v[pl.ds(c * CHUNK + s * 16, 16)]
                it = iid_v[pl.ds(c * CHUNK + s * 16, 16)]
                gu_v[c, pl.ds(s * 16, 16)] = (
                    lax.shift_right_logical(u, 16) * 8192 + (u & 8191))
                gi_v[c, pl.ds(s * 16, 16)] = (
                    lax.shift_right_logical(it, 16) * 8192 + (it & 8191))
                cu_v[pl.ds(c * CHUNK + s * 16, 16)] = (
                    (lax.shift_right_logical(u, 13) & 7) * 16)
                ci_v[pl.ds(c * CHUNK + s * 16, 16)] = (
                    (lax.shift_right_logical(it, 13) & 7) * 16)

            cp_u = pltpu.async_copy(tu_hbm.at[gu_v.at[c]], su_v, sem_u)
            cp_i = pltpu.async_copy(ti_hbm.at[gi_v.at[c]], si_v, sem_i)
            cp_u.wait()
            cp_i.wait()

            def group(g, carry2, c=c):
                for jj in range(16):
                    slot = g * 16 + jj
                    row = c * CHUNK + slot
                    cu = plsc.load_gather(cu_v, [jnp.full((16,), row,
                                                          jnp.int32)])
                    ci = plsc.load_gather(ci_v, [jnp.full((16,), row,
                                                          jnp.int32)])
                    p = plsc.load_gather(pr_v, [jnp.full((16,), row,
                                                         jnp.int32)])
                    hu = plsc.load_gather(
                        su_v, [jnp.full((16,), slot, jnp.int32), cu + iota])
                    hi = plsc.load_gather(
                        si_v, [jnp.full((16,), slot, jnp.int32), ci + iota])
                    h = hu + hi + p * w1p + b1
                    h = jnp.maximum(h, 0.0)
                    z = jnp.sum(h * w2)
                    plsc.store_scatter(z_v, [jnp.full((16,), row, jnp.int32)],
                                       jnp.full((16,), z, jnp.float32))
                return carry2

            lax.fori_loop(0, CHUNK // 16, group, 0)

        def sig(k, carry):
            z = z_v[pl.ds(k * 16, 16)]
            z_v[pl.ds(k * 16, 16)] = 1.0 / (1.0 + jnp.exp(-z - b2))
            return carry

        lax.fori_loop(0, bpw // 16, sig, 0, unroll=4)
        pltpu.sync_copy(z_v, out_hbm.at[pl.ds(base, bpw)])

    return body(user_id, item_id, price, t_u, t_i, params)


def kernel(user_id, item_id, price, user_table, item_table, W1, b1, W2, b2):
    D = user_table.shape[1]
    t_u = _precompute_packed(user_table.T, W1[0:D, :])
    t_i = _precompute_packed(item_table.T, W1[D:2 * D, :])
    params = jnp.concatenate(
        [W1[2 * D, :], b1, W2[:, 0], jnp.full((16,), b2[0], jnp.float32)])
    return _sc_gather_mlp(user_id, item_id, price, t_u, t_i, params)


# speedup vs baseline: 8.0106x; 1.0789x over previous
import functools

import jax
import jax.numpy as jnp
from jax import lax
from jax.experimental import pallas as pl
from jax.experimental.pallas import tpu as pltpu
from jax.experimental.pallas import tpu_sc as plsc

NUM_CORES = 2
NUM_SUBCORES = 16
NW = NUM_CORES * NUM_SUBCORES
CBW = 65536
STRIPE = CBW // 8
CHUNK = 128


def _precompute_body(*refs):
    xs, wb_ref, o_ref = refs[:8], refs[8], refs[9]
    x = jnp.concatenate([r[...] for r in xs], axis=0)
    o_ref[...] = jax.lax.dot_general(
        x.astype(jnp.bfloat16), wb_ref[...].astype(jnp.bfloat16),
        dimension_numbers=(((0,), (0,)), ((), ())),
        preferred_element_type=jnp.float32)


def _precompute_packed(table_t, w):
    n = table_t.shape[1]
    nblk = pl.cdiv(n, CBW)
    wb = jnp.kron(jnp.eye(8, dtype=w.dtype), w)
    last = pl.cdiv(n, STRIPE) - 1
    stripe_specs = [
        pl.BlockSpec((32, STRIPE),
                     lambda i, j=j: (0, jnp.minimum(i * 8 + j, last)))
        for j in range(8)
    ]
    return pl.pallas_call(
        _precompute_body,
        grid=(nblk,),
        in_specs=stripe_specs + [pl.BlockSpec((256, 128), lambda i: (0, 0))],
        out_specs=pl.BlockSpec((STRIPE, 128), lambda i: (i, 0)),
        out_shape=jax.ShapeDtypeStruct((nblk * STRIPE, 128), jnp.float32),
    )(*([table_t] * 8), wb)


def _sc_gather_mlp(user_id, item_id, price, t_u, t_i, params):
    B = user_id.shape[0]
    bpw = B // NW
    nchunk = bpw // CHUNK
    mesh = plsc.VectorSubcoreMesh(core_axis_name="c", subcore_axis_name="s")

    @functools.partial(
        pl.kernel,
        mesh=mesh,
        compiler_params=pltpu.CompilerParams(needs_layout_passes=False),
        out_type=jax.ShapeDtypeStruct((B,), jnp.float32),
        scratch_types=[
            pltpu.VMEM((bpw,), jnp.int32),
            pltpu.VMEM((bpw,), jnp.int32),
            pltpu.VMEM((bpw,), jnp.float32),
            pltpu.VMEM((nchunk, CHUNK), jnp.int32),
            pltpu.VMEM((nchunk, CHUNK), jnp.int32),
            pltpu.VMEM((bpw,), jnp.int32),
            pltpu.VMEM((bpw,), jnp.int32),
            pltpu.VMEM((CHUNK, 128), jnp.float32),
            pltpu.VMEM((CHUNK, 128), jnp.float32),
            pltpu.VMEM((CHUNK, 128), jnp.float32),
            pltpu.VMEM((CHUNK, 128), jnp.float32),
            pltpu.VMEM((bpw,), jnp.float32),
            pltpu.VMEM((64,), jnp.float32),
            pltpu.SemaphoreType.DMA,
            pltpu.SemaphoreType.DMA,
            pltpu.SemaphoreType.DMA,
            pltpu.SemaphoreType.DMA,
        ],
    )
    def body(uid_hbm, iid_hbm, price_hbm, tu_hbm, ti_hbm, par_hbm, out_hbm,
             uid_v, iid_v, pr_v, gu_v, gi_v, cu_v, ci_v,
             su_a, su_b, si_a, si_b, z_v, par_v,
             sem_u0, sem_u1, sem_i0, sem_i1):
        wid = lax.axis_index("s") * NUM_CORES + lax.axis_index("c")
        base = wid * bpw
        pltpu.sync_copy(uid_hbm.at[pl.ds(base, bpw)], uid_v)
        pltpu.sync_copy(iid_hbm.at[pl.ds(base, bpw)], iid_v)
        pltpu.sync_copy(price_hbm.at[pl.ds(base, bpw)], pr_v)
        pltpu.sync_copy(par_hbm, par_v)
        iota = lax.iota(jnp.int32, 16)
        w1p_s = [plsc.load_gather(par_v, [jnp.full((16,), k, jnp.int32)])
                 for k in range(16)]
        b1_s = [plsc.load_gather(par_v, [jnp.full((16,), 16 + k, jnp.int32)])
                for k in range(16)]
        w2_s = [plsc.load_gather(par_v, [jnp.full((16,), 32 + k, jnp.int32)])
                for k in range(16)]
        b2 = par_v[pl.ds(48, 16)]

        for c in range(nchunk):
            for s in range(CHUNK // 16):
                u = uid_v[pl.ds(c * CHUNK + s * 16, 16)]
                it = iid_v[pl.ds(c * CHUNK + s * 16, 16)]
                gu_v[c, pl.ds(s * 16, 16)] = (
                    lax.shift_right_logical(u, 16) * 8192 + (u & 8191))
                gi_v[c, pl.ds(s * 16, 16)] = (
                    lax.shift_right_logical(it, 16) * 8192 + (it & 8191))
                cu_v[pl.ds(c * CHUNK + s * 16, 16)] = (
                    (lax.shift_right_logical(u, 13) & 7) * 16)
                ci_v[pl.ds(c * CHUNK + s * 16, 16)] = (
                    (lax.shift_right_logical(it, 13) & 7) * 16)

        bufs = [(su_a, si_a, sem_u0, sem_i0), (su_b, si_b, sem_u1, sem_i1)]

        def fire(c):
            su, si, squ, sqi = bufs[c % 2]
            return (pltpu.async_copy(tu_hbm.at[gu_v.at[c]], su, squ),
                    pltpu.async_copy(ti_hbm.at[gi_v.at[c]], si, sqi))

        pending = fire(0)
        for c in range(nchunk):
            nxt = fire(c + 1) if c + 1 < nchunk else None
            pending[0].wait()
            pending[1].wait()
            pending = nxt
            su, si = bufs[c % 2][0], bufs[c % 2][1]
            for g in range(CHUNK // 16):
                off = c * CHUNK + g * 16
                ridx = iota + g * 16
                col_u = cu_v[pl.ds(off, 16)]
                col_i = ci_v[pl.ds(off, 16)]
                p16 = pr_v[pl.ds(off, 16)]
                acc = jnp.zeros((16,), jnp.float32)
                for k in range(16):
                    hk = (plsc.load_gather(su, [ridx, col_u + k])
                          + plsc.load_gather(si, [ridx, col_i + k])
                          + p16 * w1p_s[k] + b1_s[k])
                    acc = acc + jnp.maximum(hk, 0.0) * w2_s[k]
                z_v[pl.ds(off, 16)] = 1.0 / (1.0 + jnp.exp(-acc - b2))

        pltpu.sync_copy(z_v, out_hbm.at[pl.ds(base, bpw)])

    return body(user_id, item_id, price, t_u, t_i, params)


def kernel(user_id, item_id, price, user_table, item_table, W1, b1, W2, b2):
    D = user_table.shape[1]
    t_u = _precompute_packed(user_table.T, W1[0:D, :])
    t_i = _precompute_packed(item_table.T, W1[D:2 * D, :])
    params = jnp.concatenate(
        [W1[2 * D, :], b1, W2[:, 0], jnp.full((16,), b2[0], jnp.float32)])
    return _sc_gather_mlp(user_id, item_id, price, t_u, t_i, params)


# speedup vs baseline: 8.4156x; 1.0506x over previous
import functools

import jax
import jax.numpy as jnp
from jax import lax
from jax.experimental import pallas as pl
from jax.experimental.pallas import tpu as pltpu
from jax.experimental.pallas import tpu_sc as plsc

NUM_CORES = 2
NUM_SUBCORES = 16
NW = NUM_CORES * NUM_SUBCORES
CBW = 65536
STRIPE = CBW // 8
CHUNK = 128


def _precompute_body(*refs):
    xs, wb_ref, o_ref = refs[:8], refs[8], refs[9]
    x = jnp.concatenate([r[...] for r in xs], axis=0)
    o_ref[...] = jax.lax.dot_general(
        x.astype(jnp.bfloat16), wb_ref[...].astype(jnp.bfloat16),
        dimension_numbers=(((0,), (0,)), ((), ())),
        preferred_element_type=jnp.float32)


def _precompute_packed(table_t, w):
    n = table_t.shape[1]
    nblk = pl.cdiv(n, CBW)
    wb = jnp.kron(jnp.eye(8, dtype=w.dtype), w)
    last = pl.cdiv(n, STRIPE) - 1
    stripe_specs = [
        pl.BlockSpec((32, STRIPE),
                     lambda i, j=j: (0, jnp.minimum(i * 8 + j, last)))
        for j in range(8)
    ]
    return pl.pallas_call(
        _precompute_body,
        grid=(nblk,),
        in_specs=stripe_specs + [pl.BlockSpec((256, 128), lambda i: (0, 0))],
        out_specs=pl.BlockSpec((STRIPE, 128), lambda i: (i, 0)),
        out_shape=jax.ShapeDtypeStruct((nblk * STRIPE, 128), jnp.float32),
    )(*([table_t] * 8), wb)


def _sc_gather_mlp(user_id, item_id, price, t_u, t_i, params):
    B = user_id.shape[0]
    bpw = B // NW
    nchunk = bpw // CHUNK
    mesh = plsc.VectorSubcoreMesh(core_axis_name="c", subcore_axis_name="s")

    @functools.partial(
        pl.kernel,
        mesh=mesh,
        compiler_params=pltpu.CompilerParams(needs_layout_passes=False,
                                             use_tc_tiling_on_sc=False),
        out_type=jax.ShapeDtypeStruct((B,), jnp.float32),
        scratch_types=[
            pltpu.VMEM((bpw,), jnp.int32),
            pltpu.VMEM((bpw,), jnp.int32),
            pltpu.VMEM((bpw,), jnp.float32),
            pltpu.VMEM((nchunk, CHUNK), jnp.int32),
            pltpu.VMEM((nchunk, CHUNK), jnp.int32),
            pltpu.VMEM((CHUNK, 16), jnp.float32),
            pltpu.VMEM((CHUNK, 16), jnp.float32),
            pltpu.VMEM((CHUNK, 16), jnp.float32),
            pltpu.VMEM((CHUNK, 16), jnp.float32),
            pltpu.VMEM((bpw,), jnp.float32),
            pltpu.VMEM((64,), jnp.float32),
            pltpu.SemaphoreType.DMA,
            pltpu.SemaphoreType.DMA,
            pltpu.SemaphoreType.DMA,
            pltpu.SemaphoreType.DMA,
        ],
    )
    def body(uid_hbm, iid_hbm, price_hbm, tu_hbm, ti_hbm, par_hbm, out_hbm,
             uid_v, iid_v, pr_v, gu_v, gi_v,
             su_a, su_b, si_a, si_b, z_v, par_v,
             sem_u0, sem_u1, sem_i0, sem_i1):
        wid = lax.axis_index("s") * NUM_CORES + lax.axis_index("c")
        base = wid * bpw
        pltpu.sync_copy(uid_hbm.at[pl.ds(base, bpw)], uid_v)
        pltpu.sync_copy(iid_hbm.at[pl.ds(base, bpw)], iid_v)
        pltpu.sync_copy(price_hbm.at[pl.ds(base, bpw)], pr_v)
        pltpu.sync_copy(par_hbm, par_v)
        iota = lax.iota(jnp.int32, 16)
        w1p_s = [plsc.load_gather(par_v, [jnp.full((16,), k, jnp.int32)])
                 for k in range(16)]
        b1_s = [plsc.load_gather(par_v, [jnp.full((16,), 16 + k, jnp.int32)])
                for k in range(16)]
        w2_s = [plsc.load_gather(par_v, [jnp.full((16,), 32 + k, jnp.int32)])
                for k in range(16)]
        b2 = par_v[pl.ds(48, 16)]

        for c in range(nchunk):
            for s in range(CHUNK // 16):
                u = uid_v[pl.ds(c * CHUNK + s * 16, 16)]
                it = iid_v[pl.ds(c * CHUNK + s * 16, 16)]
                gu_v[c, pl.ds(s * 16, 16)] = (
                    (lax.shift_right_logical(u, 16) * 8192 + (u & 8191)) * 8
                    + (lax.shift_right_logical(u, 13) & 7))
                gi_v[c, pl.ds(s * 16, 16)] = (
                    (lax.shift_right_logical(it, 16) * 8192 + (it & 8191)) * 8
                    + (lax.shift_right_logical(it, 13) & 7))

        bufs = [(su_a, si_a, sem_u0, sem_i0), (su_b, si_b, sem_u1, sem_i1)]

        def fire(c):
            su, si, squ, sqi = bufs[c % 2]
            return (pltpu.async_copy(tu_hbm.at[gu_v.at[c]], su, squ),
                    pltpu.async_copy(ti_hbm.at[gi_v.at[c]], si, sqi))

        pending = fire(0)
        for c in range(nchunk):
            nxt = fire(c + 1) if c + 1 < nchunk else None
            pending[0].wait()
            pending[1].wait()
            pending = nxt
            su, si = bufs[c % 2][0], bufs[c % 2][1]
            for g in range(CHUNK // 16):
                off = c * CHUNK + g * 16
                ridx = iota + g * 16
                p16 = pr_v[pl.ds(off, 16)]
                acc = jnp.zeros((16,), jnp.float32)
                for k in range(16):
                    kidx = jnp.full((16,), k, jnp.int32)
                    hk = (plsc.load_gather(su, [ridx, kidx])
                          + plsc.load_gather(si, [ridx, kidx])
                          + p16 * w1p_s[k] + b1_s[k])
                    acc = acc + jnp.maximum(hk, 0.0) * w2_s[k]
                z_v[pl.ds(off, 16)] = 1.0 / (1.0 + jnp.exp(-acc - b2))

        pltpu.sync_copy(z_v, out_hbm.at[pl.ds(base, bpw)])

    return body(user_id, item_id, price, t_u, t_i, params)


def kernel(user_id, item_id, price, user_table, item_table, W1, b1, W2, b2):
    D = user_table.shape[1]
    t_u = _precompute_packed(user_table.T, W1[0:D, :]).reshape(-1, 16)
    t_i = _precompute_packed(item_table.T, W1[D:2 * D, :]).reshape(-1, 16)
    params = jnp.concatenate(
        [W1[2 * D, :], b1, W2[:, 0], jnp.full((16,), b2[0], jnp.float32)])
    return _sc_gather_mlp(user_id, item_id, price, t_u, t_i, params)
